# baseline (speedup 1.0000x reference)
import functools

import numpy as np
import jax
import jax.numpy as jnp
from jax import lax
from jax.experimental import pallas as pl
from jax.experimental.pallas import tpu as pltpu
from jax.experimental.pallas import tpu_sc as plsc

B = 4096
F = 26
V = 100000
DE = 64
NODES = F + 1

_li, _lj = np.tril_indices(NODES, -1)
_y_mask = _lj == 0
_P_Y = np.nonzero(_y_mask)[0]
_P_EE = np.nonzero(~_y_mask)[0]
_COLS_EE = (_li[~_y_mask] - 1) * F + (_lj[~_y_mask] - 1)

NC, NS = 2, 16
NW = NC * NS
ROWS = B * F
RPW = ROWS // NW
CH = 128
NCH = RPW // CH


def _gather_body(table_hbm, idx_hbm, out_hbm, idx_v, buf, gsem):
    wid = lax.axis_index("s") * NC + lax.axis_index("c")
    base = wid * RPW
    pltpu.sync_copy(idx_hbm.at[wid], idx_v)

    def chunk(c, carry):
        pltpu.async_copy(table_hbm.at[idx_v.at[c]], buf, gsem).wait()
        pltpu.sync_copy(buf, out_hbm.at[pl.ds(base + c * CH, CH)])
        return carry

    lax.fori_loop(0, NCH, chunk, 0)


@functools.partial(jax.jit, donate_argnums=())
def _sc_gather(table, idx2d):
    mesh = plsc.VectorSubcoreMesh(core_axis_name="c", subcore_axis_name="s")
    return pl.kernel(
        _gather_body,
        out_type=jax.ShapeDtypeStruct((ROWS, DE), jnp.float32),
        mesh=mesh,
        scratch_types=[
            pltpu.VMEM((NCH, CH), jnp.int32),
            pltpu.VMEM((CH, DE), jnp.float32),
            pltpu.SemaphoreType.DMA,
        ],
        compiler_params=pltpu.CompilerParams(use_tc_tiling_on_sc=False),
    )(table, idx2d)


def _dense_body(x_ref, e_ref, w0, b0, w1, b1, w2, b2, w3, b3,
                wy, wye, wee, tb0r, w5, b5, w6, b6, out_ref):
    def lin(h, w, b):
        return lax.dot_general(h, w[...], (((1,), (1,)), ((), ()))) + b[...]

    x = x_ref[...]
    e = e_ref[...]
    y = jnp.maximum(lin(x, w0, b0), 0)
    y = jnp.maximum(lin(y, w1, b1), 0)
    y = jnp.maximum(lin(y, w2, b2), 0)
    y = jnp.maximum(lin(y, w3, b3), 0)
    zye = jnp.sum(e * y[:, None, :], axis=2)
    zee = lax.dot_general(e, e, (((2,), (2,)), ((0,), (0,))))
    zee_f = zee.reshape(zee.shape[0], F * F)
    h = (lax.dot_general(y, wy[...], (((1,), (1,)), ((), ())))
         + lax.dot_general(zye, wye[...], (((1,), (1,)), ((), ())))
         + lax.dot_general(zee_f, wee[...], (((1,), (1,)), ((), ())))
         + tb0r[...])
    h = jnp.maximum(h, 0)
    h = jnp.maximum(lin(h, w5, b5), 0)
    o = jnp.sum(h * w6[...], axis=1, keepdims=True) + b6[0, 0]
    out_ref[...] = jax.nn.sigmoid(o)


def _dense_forward(x, e3, w0, b0, w1, b1, w2, b2, w3, b3,
                   wy, wye, wee, tb0, w5, b5, w6, b6, block_b=512):
    nblk = B // block_b
    full = lambda a: pl.BlockSpec(a.shape, lambda i: (0,) * a.ndim)
    args = (w0, b0, w1, b1, w2, b2, w3, b3, wy, wye, wee, tb0, w5, b5, w6, b6)
    return pl.pallas_call(
        _dense_body,
        grid=(nblk,),
        in_specs=[
            pl.BlockSpec((block_b, x.shape[1]), lambda i: (i, 0)),
            pl.BlockSpec((block_b, F, DE), lambda i: (i, 0, 0)),
            *[full(a) for a in args],
        ],
        out_specs=pl.BlockSpec((block_b, 1), lambda i: (i, 0)),
        out_shape=jax.ShapeDtypeStruct((B, 1), jnp.float32),
    )(x, e3, *args)


def kernel(dense_x, sparse_idx, emb_tables, bw0, bb0, bw1, bb1, bw2, bb2,
           bw3, bb3, tw0, tb0, tw1, tb1, tw2, tb2):
    table = emb_tables.reshape(F * V, DE)
    idx = (sparse_idx.astype(jnp.int32).T
           + (jnp.arange(F, dtype=jnp.int32) * V)[None, :])
    idx2d = idx.reshape(NW, NCH, CH)
    emb_flat = _sc_gather(table, idx2d)
    e3 = emb_flat.reshape(B, F, DE)

    wy = tw0[:, :DE]
    wye = tw0[:, DE + _P_Y]
    wee = jnp.zeros((tw0.shape[0], F * F), tw0.dtype).at[:, _COLS_EE].set(
        tw0[:, DE + _P_EE])

    return _dense_forward(
        dense_x, e3, bw0, bb0[None, :], bw1, bb1[None, :], bw2, bb2[None, :],
        bw3, bb3[None, :], wy, wye, wee, tb0[None, :], tw1, tb1[None, :],
        tw2, tb2[None, :])


# speedup vs baseline: 1.0086x; 1.0086x over previous
import functools

import numpy as np
import jax
import jax.numpy as jnp
from jax import lax
from jax.experimental import pallas as pl
from jax.experimental.pallas import tpu as pltpu
from jax.experimental.pallas import tpu_sc as plsc

B = 4096
F = 26
V = 100000
DE = 64
NODES = F + 1

_li, _lj = np.tril_indices(NODES, -1)
_y_mask = _lj == 0
_P_Y = np.nonzero(_y_mask)[0]
_P_EE = np.nonzero(~_y_mask)[0]
_COLS_EE = (_li[~_y_mask] - 1) * F + (_lj[~_y_mask] - 1)

NC, NS = 2, 16
NW = NC * NS
ROWS = B * F
RPW = ROWS // NW
CH = 128
NCH = RPW // CH


JPF = B // CH


def _gather_body(table_hbm, idx_hbm, oidx_hbm, out_hbm, idx_v, oidx_v, buf,
                 gsem, ssem):
    wid = lax.axis_index("s") * NC + lax.axis_index("c")
    pltpu.sync_copy(idx_hbm.at[wid], idx_v)
    pltpu.sync_copy(oidx_hbm.at[wid], oidx_v)

    def chunk(c, carry):
        g = wid * NCH + c
        f = g // JPF
        pltpu.async_copy(table_hbm.at[f].at[idx_v.at[c]], buf, gsem).wait()
        pltpu.async_copy(buf, out_hbm.at[oidx_v.at[c]], ssem).wait()
        return carry

    lax.fori_loop(0, NCH, chunk, 0)


def _sc_gather(table3, idx3, oidx3):
    mesh = plsc.VectorSubcoreMesh(core_axis_name="c", subcore_axis_name="s")
    return pl.kernel(
        _gather_body,
        out_type=jax.ShapeDtypeStruct((ROWS, DE), jnp.float32),
        mesh=mesh,
        scratch_types=[
            pltpu.VMEM((NCH, CH), jnp.int32),
            pltpu.VMEM((NCH, CH), jnp.int32),
            pltpu.VMEM((CH, DE), jnp.float32),
            pltpu.SemaphoreType.DMA,
            pltpu.SemaphoreType.DMA,
        ],
        compiler_params=pltpu.CompilerParams(use_tc_tiling_on_sc=False),
    )(table3, idx3, oidx3)


def _dense_body(x_ref, e_ref, w0, b0, w1, b1, w2, b2, w3, b3,
                wy, wye, wee, tb0r, w5, b5, w6, b6, out_ref):
    def lin(h, w, b):
        return lax.dot_general(h, w[...], (((1,), (1,)), ((), ()))) + b[...]

    x = x_ref[...]
    e = e_ref[...]
    y = jnp.maximum(lin(x, w0, b0), 0)
    y = jnp.maximum(lin(y, w1, b1), 0)
    y = jnp.maximum(lin(y, w2, b2), 0)
    y = jnp.maximum(lin(y, w3, b3), 0)
    zye = jnp.sum(e * y[:, None, :], axis=2)
    zee = lax.dot_general(e, e, (((2,), (2,)), ((0,), (0,))))
    zee_f = zee.reshape(zee.shape[0], F * F)
    h = (lax.dot_general(y, wy[...], (((1,), (1,)), ((), ())))
         + lax.dot_general(zye, wye[...], (((1,), (1,)), ((), ())))
         + lax.dot_general(zee_f, wee[...], (((1,), (1,)), ((), ())))
         + tb0r[...])
    h = jnp.maximum(h, 0)
    h = jnp.maximum(lin(h, w5, b5), 0)
    o = jnp.sum(h * w6[...], axis=1, keepdims=True) + b6[0, 0]
    out_ref[...] = jax.nn.sigmoid(o)


def _dense_forward(x, e3, w0, b0, w1, b1, w2, b2, w3, b3,
                   wy, wye, wee, tb0, w5, b5, w6, b6, block_b=512):
    nblk = B // block_b
    full = lambda a: pl.BlockSpec(a.shape, lambda i: (0,) * a.ndim)
    args = (w0, b0, w1, b1, w2, b2, w3, b3, wy, wye, wee, tb0, w5, b5, w6, b6)
    return pl.pallas_call(
        _dense_body,
        grid=(nblk,),
        in_specs=[
            pl.BlockSpec((block_b, x.shape[1]), lambda i: (i, 0)),
            pl.BlockSpec((block_b, F, DE), lambda i: (i, 0, 0)),
            *[full(a) for a in args],
        ],
        out_specs=pl.BlockSpec((block_b, 1), lambda i: (i, 0)),
        out_shape=jax.ShapeDtypeStruct((B, 1), jnp.float32),
    )(x, e3, *args)


def kernel(dense_x, sparse_idx, emb_tables, bw0, bb0, bw1, bb1, bw2, bb2,
           bw3, bb3, tw0, tb0, tw1, tb1, tw2, tb2):
    idx3 = sparse_idx.astype(jnp.int32).reshape(NW, NCH, CH)
    g = np.arange(NW * NCH)
    orows = ((g % JPF)[:, None] * CH + np.arange(CH)[None, :]) * F \
        + (g // JPF)[:, None]
    oidx3 = jnp.asarray(orows.reshape(NW, NCH, CH), dtype=jnp.int32)
    emb_flat = _sc_gather(emb_tables, idx3, oidx3)
    e3 = emb_flat.reshape(B, F, DE)

    wy = tw0[:, :DE]
    wye = tw0[:, DE + _P_Y]
    wee = jnp.zeros((tw0.shape[0], F * F), tw0.dtype).at[:, _COLS_EE].set(
        tw0[:, DE + _P_EE])

    return _dense_forward(
        dense_x, e3, bw0, bb0[None, :], bw1, bb1[None, :], bw2, bb2[None, :],
        bw3, bb3[None, :], wy, wye, wee, tb0[None, :], tw1, tb1[None, :],
        tw2, tb2[None, :])
